# baseline (speedup 1.0000x reference)
import functools

import jax
import jax.numpy as jnp
from jax import lax
from jax.experimental import pallas as pl
from jax.experimental.pallas import tpu as pltpu
from jax.experimental.pallas import tpu_sc as plsc

VOCAB = 457
EMB = 64
BATCH = 16384
SEQ_LEN = 200
N = BATCH * SEQ_LEN
LINES = N // 2
PIDX_ROWS = LINES // 128

_info = plsc.get_sparse_core_info()
NC = _info.num_cores
NS = _info.num_subcores
NW = NC * NS
SUB = 128
NSUB = 2
CHUNK_LINES = SUB * NSUB
PER_W_LINES = LINES // NW
N_ITERS = PER_W_LINES // CHUNK_LINES

assert LINES % (NW * CHUNK_LINES) == 0 and N_ITERS % 2 == 0


def _sc_lookup_one(pidx, pair_table):
    mesh = plsc.VectorSubcoreMesh(core_axis_name="c", subcore_axis_name="s")

    @functools.partial(
        pl.kernel,
        mesh=mesh,
        out_type=jax.ShapeDtypeStruct((LINES, 128), jnp.float32),
        scratch_types=[
            pltpu.VMEM((2, NSUB, SUB), jnp.int32),
            pltpu.VMEM((2, CHUNK_LINES, 128), jnp.float32),
            [pltpu.SemaphoreType.DMA, pltpu.SemaphoreType.DMA],
            [pltpu.SemaphoreType.DMA, pltpu.SemaphoreType.DMA],
        ],
        compiler_params=pltpu.CompilerParams(use_tc_tiling_on_sc=False),
    )
    def k(idx_hbm, tab_hbm, out_hbm, idx_v, rows_v, gsem, osem):
        wid = lax.axis_index("s") * NC + lax.axis_index("c")
        base_irow = wid * (PER_W_LINES // SUB)
        base_line = wid * PER_W_LINES

        def fire_gathers(g, b):
            irow = base_irow + g * NSUB
            pltpu.sync_copy(idx_hbm.at[pl.ds(irow, NSUB)], idx_v.at[b])
            for j in range(NSUB):
                pltpu.async_copy(
                    tab_hbm.at[idx_v.at[b, j]],
                    rows_v.at[b, pl.ds(j * SUB, SUB)],
                    gsem[b],
                )

        def wait_gathers(b):
            for j in range(NSUB):
                pltpu.make_async_copy(
                    tab_hbm.at[idx_v.at[b, j]],
                    rows_v.at[b, pl.ds(j * SUB, SUB)],
                    gsem[b],
                ).wait()

        def fire_out(g, b):
            line = base_line + g * CHUNK_LINES
            pltpu.async_copy(rows_v.at[b],
                             out_hbm.at[pl.ds(line, CHUNK_LINES)], osem[b])

        def wait_out(g, b):
            line = base_line + g * CHUNK_LINES
            pltpu.make_async_copy(rows_v.at[b],
                                  out_hbm.at[pl.ds(line, CHUNK_LINES)],
                                  osem[b]).wait()

        fire_gathers(0, 0)

        def step(g2, carry):
            for b in range(2):
                g = 2 * g2 + b
                nb2 = 1 - b

                @pl.when(g + 1 < N_ITERS)
                def _():
                    @pl.when(g >= 1)
                    def _():
                        wait_out(g - 1, nb2)
                    fire_gathers(g + 1, nb2)

                wait_gathers(b)
                fire_out(g, b)
            return carry

        lax.fori_loop(0, N_ITERS // 2, step, 0)
        wait_out(N_ITERS - 1, (N_ITERS - 1) % 2)
        wait_out(N_ITERS - 2, (N_ITERS - 2) % 2)

    return k(pidx, pair_table)


def _pair_idx(a):
    p = a.astype(jnp.int32).reshape(LINES, 2)
    return (p[:, 0] * VOCAB + p[:, 1]).reshape(PIDX_ROWS, 128)


def _pair_table(tab):
    left = jnp.broadcast_to(tab[:, None, :], (VOCAB, VOCAB, EMB))
    right = jnp.broadcast_to(tab[None, :, :], (VOCAB, VOCAB, EMB))
    return jnp.concatenate([left, right], axis=-1).reshape(VOCAB * VOCAB, 128)


def _unlines(lines):
    return jnp.minimum(lines, jnp.float32(3.0e38)).reshape(BATCH, SEQ_LEN, EMB)


def kernel(seqs, exps, table_seq, table_exp):
    lines1 = _sc_lookup_one(_pair_idx(seqs), _pair_table(table_seq))
    lines2 = _sc_lookup_one(_pair_idx(exps), _pair_table(table_exp))
    return (_unlines(lines1), _unlines(lines2))
